# baseline (speedup 1.0000x reference)
import jax
import jax.numpy as jnp
from jax.experimental import pallas as pl

N_ROIS = 268
D_MODEL = 64
BATCH = 4096
RB = 4


def _bcast_kernel(tab_ref, out_ref):
    out_ref[...] = jnp.broadcast_to(tab_ref[...], out_ref.shape)


def kernel(batch_size, pos_embedding):
    tab3 = pos_embedding.reshape(N_ROIS, D_MODEL, 1)
    out = pl.pallas_call(
        _bcast_kernel,
        grid=(N_ROIS // RB,),
        in_specs=[pl.BlockSpec((RB, D_MODEL, 1), lambda i: (i, 0, 0))],
        out_specs=pl.BlockSpec((RB, D_MODEL, BATCH), lambda i: (i, 0, 0)),
        out_shape=jax.ShapeDtypeStruct((N_ROIS, D_MODEL, BATCH), jnp.float32),
    )(tab3)
    return jnp.transpose(out, (2, 0, 1))


# speedup vs baseline: 1.0636x; 1.0636x over previous
import jax
import jax.numpy as jnp
from jax.experimental import pallas as pl

N_ROIS = 268
D_MODEL = 64
BATCH = 4096
RB = 24


def _bcast_kernel(tab_ref, out_ref):
    out_ref[...] = jnp.broadcast_to(tab_ref[...], out_ref.shape)


def kernel(batch_size, pos_embedding):
    tab3 = pos_embedding.reshape(N_ROIS, D_MODEL, 1)
    out = pl.pallas_call(
        _bcast_kernel,
        grid=((N_ROIS + RB - 1) // RB,),
        in_specs=[pl.BlockSpec((RB, D_MODEL, 1), lambda i: (i, 0, 0))],
        out_specs=pl.BlockSpec((RB, D_MODEL, BATCH), lambda i: (i, 0, 0)),
        out_shape=jax.ShapeDtypeStruct((N_ROIS, D_MODEL, BATCH), jnp.float32),
    )(tab3)
    return jnp.transpose(out, (2, 0, 1))
